# baseline (speedup 1.0000x reference)
import functools

import jax
import jax.numpy as jnp
from jax.experimental import pallas as pl


_D = 262144
_N = 32
_K1 = 15
_R = (_D * _N) // 128
_G = 128 // _N


def _mkrum_kernel(y_ref, out_ref):
    y = y_ref[...]

    p = jax.lax.dot_general(
        y, y, (((0,), (0,)), ((), ())), preferred_element_type=jnp.float32
    )
    g = (
        p[0:32, 0:32]
        + p[32:64, 32:64]
        + p[64:96, 64:96]
        + p[96:128, 96:128]
    )

    rio = jax.lax.broadcasted_iota(jnp.int32, (_N, _N), 0)
    cio = jax.lax.broadcasted_iota(jnp.int32, (_N, _N), 1)
    eye = rio == cio
    zero = jnp.zeros_like(g)
    sq_col = jnp.sum(jnp.where(eye, g, zero), axis=1, keepdims=True)
    sq_row = jnp.sum(jnp.where(eye, g, zero), axis=0, keepdims=True)
    d2 = sq_col + sq_row - 2.0 * g
    cd = jnp.sqrt(jnp.maximum(d2, 0.0))

    vals = cd
    rowmask = jnp.zeros((_N, _N), jnp.float32)
    s15 = jnp.zeros((_N, 1), jnp.float32)
    big = jnp.float32(3.0e38)
    for _ in range(_K1):
        m = jnp.min(vals, axis=1, keepdims=True)
        s15 = s15 + m
        is_min = vals <= m
        fidx = jnp.min(jnp.where(is_min, cio, _N), axis=1, keepdims=True)
        sel = cio == fidx
        rowmask = jnp.where(sel, 1.0, rowmask)
        vals = jnp.where(sel, big, vals)

    mn = jnp.min(s15)
    rio1 = jax.lax.broadcasted_iota(jnp.int32, (_N, 1), 0)
    istar = jnp.min(jnp.where(s15 <= mn, rio1, _N))

    msel = jnp.where(rio == istar, rowmask, zero)
    ones = jnp.ones((_N, 1), jnp.float32)
    mcol = jax.lax.dot_general(
        msel, ones, (((0,), (0,)), ((), ())), preferred_element_type=jnp.float32
    )

    mcol128 = jnp.concatenate([mcol] * _G, axis=0)
    rio4 = jax.lax.broadcasted_iota(jnp.int32, (128, _G), 0)
    cio4 = jax.lax.broadcasted_iota(jnp.int32, (128, _G), 1)
    mmat = jnp.where(
        (rio4 // _N) == cio4, mcol128 * (1.0 / _K1), jnp.float32(0.0)
    )

    out_ref[...] = jax.lax.dot_general(
        mmat, y, (((0,), (1,)), ((), ())), preferred_element_type=jnp.float32
    )


@jax.jit
def kernel(input):
    y = jnp.reshape(input, (_R, 128))
    out_t = pl.pallas_call(
        _mkrum_kernel,
        out_shape=jax.ShapeDtypeStruct((_G, _R), jnp.float32),
    )(y)
    return jnp.reshape(jnp.transpose(out_t), (1, _D, 1))


# speedup vs baseline: 1.0543x; 1.0543x over previous
import jax
import jax.numpy as jnp
from jax.experimental import pallas as pl
from jax.experimental.pallas import tpu as pltpu


_D = 262144
_N = 32
_K1 = 15
_R = (_D * _N) // 128
_G = 128 // _N
_BK = 8192
_NB1 = _R // _BK
_RB = 8192
_NB2 = _R // _RB


def _selection_matrix(p):
    g = (
        p[0:32, 0:32]
        + p[32:64, 32:64]
        + p[64:96, 64:96]
        + p[96:128, 96:128]
    )

    rio = jax.lax.broadcasted_iota(jnp.int32, (_N, _N), 0)
    cio = jax.lax.broadcasted_iota(jnp.int32, (_N, _N), 1)
    eye = rio == cio
    zero = jnp.zeros_like(g)
    sq_col = jnp.sum(jnp.where(eye, g, zero), axis=1, keepdims=True)
    sq_row = jnp.sum(jnp.where(eye, g, zero), axis=0, keepdims=True)
    d2 = sq_col + sq_row - 2.0 * g
    cd = jnp.sqrt(jnp.maximum(d2, 0.0))

    vals = cd
    rowmask = jnp.zeros((_N, _N), jnp.float32)
    s15 = jnp.zeros((_N, 1), jnp.float32)
    big = jnp.float32(3.0e38)
    for _ in range(_K1):
        m = jnp.min(vals, axis=1, keepdims=True)
        s15 = s15 + m
        is_min = vals <= m
        fidx = jnp.min(jnp.where(is_min, cio, _N), axis=1, keepdims=True)
        sel = cio == fidx
        rowmask = jnp.where(sel, 1.0, rowmask)
        vals = jnp.where(sel, big, vals)

    mn = jnp.min(s15)
    rio1 = jax.lax.broadcasted_iota(jnp.int32, (_N, 1), 0)
    istar = jnp.min(jnp.where(s15 <= mn, rio1, _N))

    msel = jnp.where(rio == istar, rowmask, zero)
    ones = jnp.ones((_N, 1), jnp.float32)
    mcol = jax.lax.dot_general(
        msel, ones, (((0,), (0,)), ((), ())), preferred_element_type=jnp.float32
    )

    mcol128 = jnp.concatenate([mcol] * _G, axis=0)
    rio4 = jax.lax.broadcasted_iota(jnp.int32, (128, _G), 0)
    cio4 = jax.lax.broadcasted_iota(jnp.int32, (128, _G), 1)
    return jnp.where(
        (rio4 // _N) == cio4, mcol128 * (1.0 / _K1), jnp.float32(0.0)
    )


def _mkrum_kernel(y_ref, out_ref, ysave_ref, pacc_ref, mmat_ref):
    i = pl.program_id(0)

    @pl.when(i < _NB1)
    def _phase1():
        yblk = y_ref[...]
        part = jax.lax.dot_general(
            yblk, yblk, (((0,), (0,)), ((), ())),
            preferred_element_type=jnp.float32,
        )

        @pl.when(i == 0)
        def _():
            pacc_ref[...] = part

        @pl.when(i > 0)
        def _():
            pacc_ref[...] = pacc_ref[...] + part

        ysave_ref[pl.ds(i * _BK, _BK), :] = yblk

    @pl.when(i == _NB1)
    def _boundary():
        mmat_ref[...] = _selection_matrix(pacc_ref[...])

    @pl.when(i >= _NB1)
    def _phase2():
        j = i - _NB1
        yb = ysave_ref[pl.ds(j * _RB, _RB), :]
        out_ref[...] = jax.lax.dot_general(
            yb, mmat_ref[...], (((1,), (0,)), ((), ())),
            preferred_element_type=jnp.float32,
        )


@jax.jit
def kernel(input):
    y = jnp.reshape(input, (_R, 128))
    out = pl.pallas_call(
        _mkrum_kernel,
        grid=(_NB1 + _NB2,),
        in_specs=[
            pl.BlockSpec((_BK, 128), lambda i: (jnp.minimum(i, _NB1 - 1), 0)),
        ],
        out_specs=pl.BlockSpec((_RB, _G), lambda i: (jnp.maximum(i - _NB1, 0), 0)),
        out_shape=jax.ShapeDtypeStruct((_R, _G), jnp.float32),
        scratch_shapes=[
            pltpu.VMEM((_R, 128), jnp.float32),
            pltpu.VMEM((128, 128), jnp.float32),
            pltpu.VMEM((128, _G), jnp.float32),
        ],
    )(y)
    return jnp.reshape(out, (1, _D, 1))
